# baseline (speedup 1.0000x reference)
import functools

import jax
import jax.numpy as jnp
from jax import lax
from jax.experimental import pallas as pl
from jax.experimental.pallas import tpu as pltpu
from jax.experimental.pallas import tpu_sc as plsc

_NC = 2
_NS = 16
_NW = _NC * _NS
_CH = 128


def _sc_gather(center_emb, context_emb, cid, xid, nid, B, K, D):
    cpw = B // _NW // _CH
    npw = B * K // _NW // _CH
    nsc = 2
    npsc = npw // nsc
    rows_w = B // _NW
    nrows_sc = npsc * _CH

    mesh = plsc.VectorSubcoreMesh(core_axis_name="c", subcore_axis_name="s")

    @functools.partial(
        pl.kernel,
        mesh=mesh,
        out_type=[
            jax.ShapeDtypeStruct((B, D), jnp.float32),
            jax.ShapeDtypeStruct((B, D), jnp.float32),
            jax.ShapeDtypeStruct((B * K, D), jnp.float32),
        ],
        scratch_types=[
            pltpu.VMEM((cpw, _CH), jnp.int32),
            pltpu.VMEM((npw, _CH), jnp.int32),
            pltpu.VMEM((rows_w, D), jnp.float32),
            pltpu.VMEM((nrows_sc, D), jnp.float32),
            pltpu.SemaphoreType.DMA,
        ],
        compiler_params=pltpu.CompilerParams(use_tc_tiling_on_sc=False),
    )
    def gather(cemb, xemb, cids, xids, nids, c_out, x_out, n_out,
               idxc_v, idxn_v, rows_v, nrows_v, sem):
        w = lax.axis_index("s") * _NC + lax.axis_index("c")

        pltpu.sync_copy(cids.at[w], idxc_v)
        hs = [pltpu.async_copy(cemb.at[idxc_v.at[j]],
                               rows_v.at[pl.ds(j * _CH, _CH)], sem)
              for j in range(cpw)]
        for h in hs:
            h.wait()
        pltpu.sync_copy(rows_v, c_out.at[pl.ds(w * rows_w, rows_w)])

        pltpu.sync_copy(xids.at[w], idxc_v)
        hs = [pltpu.async_copy(xemb.at[idxc_v.at[j]],
                               rows_v.at[pl.ds(j * _CH, _CH)], sem)
              for j in range(cpw)]
        for h in hs:
            h.wait()
        pltpu.sync_copy(rows_v, x_out.at[pl.ds(w * rows_w, rows_w)])

        pltpu.sync_copy(nids.at[w], idxn_v)
        for s in range(nsc):
            hs = [pltpu.async_copy(xemb.at[idxn_v.at[s * npsc + j]],
                                   nrows_v.at[pl.ds(j * _CH, _CH)], sem)
                  for j in range(npsc)]
            for h in hs:
                h.wait()
            pltpu.sync_copy(
                nrows_v,
                n_out.at[pl.ds(w * npw * _CH + s * nrows_sc, nrows_sc)])

    return gather(center_emb, context_emb, cid, xid, nid)


def _selu(v):
    return 1.0507009873554805 * jnp.where(
        v > 0, v, 1.6732632423543772 * (jnp.exp(v) - 1.0))


def _softplus(z):
    return jnp.maximum(z, 0.0) + jnp.log(1.0 + jnp.exp(-jnp.abs(z)))


def _tc_loss_body(K, cr, xr, nr, wc, bc, wx, bx, out):
    i = pl.program_id(0)
    bt = cr.shape[0]
    d = cr.shape[1]
    dn = (((1,), (1,)), ((), ()))
    c = _selu(lax.dot_general(cr[...], wc[...], dn,
                              preferred_element_type=jnp.float32) + bc[...])
    x = _selu(lax.dot_general(xr[...], wx[...], dn,
                              preferred_element_type=jnp.float32) + bx[...])
    n = _selu(lax.dot_general(nr[...], wx[...], dn,
                              preferred_element_type=jnp.float32) + bx[...])
    pos = jnp.sum(c * x, axis=1, keepdims=True)
    pos = jnp.clip(pos, -10.0, 10.0)
    pos_loss = jnp.sum(_softplus(-pos))
    n3 = n.reshape(bt, K, d)
    neg = jnp.sum(n3 * c[:, None, :], axis=2)
    neg = jnp.clip(neg, -10.0, 10.0)
    neg_loss = jnp.sum(_softplus(neg))

    @pl.when(i == 0)
    def _():
        out[0, 0] = 0.0

    out[0, 0] += pos_loss + neg_loss


def _tc_loss(crows, xrows, nrows, wc, bc, wx, bx, B, K, D):
    bt = 2048
    grid = (B // bt,)
    return pl.pallas_call(
        functools.partial(_tc_loss_body, K),
        grid=grid,
        in_specs=[
            pl.BlockSpec((bt, D), lambda i: (i, 0)),
            pl.BlockSpec((bt, D), lambda i: (i, 0)),
            pl.BlockSpec((bt * K, D), lambda i: (i, 0)),
            pl.BlockSpec((D, D), lambda i: (0, 0)),
            pl.BlockSpec((1, D), lambda i: (0, 0)),
            pl.BlockSpec((D, D), lambda i: (0, 0)),
            pl.BlockSpec((1, D), lambda i: (0, 0)),
        ],
        out_specs=pl.BlockSpec((1, 1), lambda i: (0, 0),
                               memory_space=pltpu.SMEM),
        out_shape=jax.ShapeDtypeStruct((1, 1), jnp.float32),
        compiler_params=pltpu.CompilerParams(
            dimension_semantics=("arbitrary",)),
    )(crows, xrows, nrows, wc, bc, wx, bx)


def kernel(center_ids, context_ids, neg_context_ids, center_emb, context_emb,
           W_center, b_center, W_context, b_context):
    B = center_ids.shape[0]
    K = neg_context_ids.shape[1]
    D = center_emb.shape[1]
    cid = center_ids.astype(jnp.int32).reshape(_NW, B // _NW // _CH, _CH)
    xid = context_ids.astype(jnp.int32).reshape(_NW, B // _NW // _CH, _CH)
    nid = neg_context_ids.astype(jnp.int32).reshape(
        _NW, B * K // _NW // _CH, _CH)
    crows, xrows, nrows = _sc_gather(center_emb, context_emb, cid, xid, nid,
                                     B, K, D)
    total = _tc_loss(crows, xrows, nrows,
                     W_center, b_center.reshape(1, D),
                     W_context, b_context.reshape(1, D), B, K, D)
    return total[0, 0] / B
